# Initial kernel scaffold; baseline (speedup 1.0000x reference)
import jax
import jax.numpy as jnp
from jax.experimental import pallas as pl
from jax.experimental.pallas import tpu as pltpu
from jax.experimental.pallas import tpu_sc as plsc

NUM_ACT = 96
SIZE = 257
INV_GRID = 32.0
HALF = 128

ROW_LEN = 224 * 224
LANES = 16
CHUNK = 3584
CHUNKS_PER_ROW = ROW_LEN // CHUNK


def kernel(x, coefficients_vect):
    b, c, h, w = x.shape
    rows = b * c
    n = rows * ROW_LEN
    x_flat = x.reshape(n)
    base = (jnp.arange(rows, dtype=jnp.int32) % NUM_ACT) * SIZE
    base_arr = jnp.broadcast_to(base[:, None], (rows, LANES))

    mesh = plsc.VectorSubcoreMesh(core_axis_name="core", subcore_axis_name="subcore")

    @pl.kernel(
        out_type=jax.ShapeDtypeStruct((n,), jnp.float32),
        mesh=mesh,
        scratch_types=[pltpu.VMEM((NUM_ACT * SIZE,), jnp.float32)],
    )
    def spline_kernel(x_hbm, base_hbm, coeff_hbm, o_hbm, tab_v):
        pltpu.sync_copy(coeff_hbm, tab_v)

        def body(x_vmem, base_vmem, o_vmem):
            bvec = base_vmem[0, :]

            @pl.loop(0, CHUNK, step=LANES)
            def _(k):
                xv = x_vmem[pl.ds(k, LANES)]
                wv = xv * INV_GRID + jnp.float32(HALF)
                z = jnp.minimum(jnp.maximum(wv, 0.0), jnp.float32(2 * HALF - 1))
                i = z.astype(jnp.int32)
                frac = wv - i.astype(jnp.float32)
                idx = i + bvec
                c0 = plsc.load_gather(tab_v, [idx])
                c1 = plsc.load_gather(tab_v, [idx + 1])
                o_vmem[pl.ds(k, LANES)] = c0 + frac * (c1 - c0)

        pltpu.emit_pipeline(
            body,
            grid=(rows * CHUNKS_PER_ROW,),
            in_specs=[
                pl.BlockSpec((CHUNK,), lambda j: (j,)),
                pl.BlockSpec((1, LANES), lambda j: (j // CHUNKS_PER_ROW, 0)),
            ],
            out_specs=[pl.BlockSpec((CHUNK,), lambda j: (j,))],
            core_axis_name=("core", "subcore"),
            dimension_semantics=(pltpu.PARALLEL,),
        )(x_hbm, base_hbm, o_hbm)

    out = spline_kernel(x_flat, base_arr, coefficients_vect)
    return out.reshape(b, c, h, w)


# speedup vs baseline: 366.4787x; 366.4787x over previous
import dataclasses

import jax
import jax.numpy as jnp
from jax.experimental import pallas as pl
from jax.experimental.pallas import tpu as pltpu
from jax.experimental.pallas import tpu_sc as plsc

NUM_ACT = 96
SIZE = 257
INV_GRID = 32.0
HALF = 128

ROW_LEN = 224 * 224
LANES = 16
CHUNK = 3584
CHUNKS_PER_ROW = ROW_LEN // CHUNK


def kernel(x, coefficients_vect):
    b, c, h, w = x.shape
    rows = b * c
    n = rows * ROW_LEN
    x_flat = x.reshape(n)
    base = (jnp.arange(rows, dtype=jnp.int32) % NUM_ACT) * SIZE
    base_arr = jnp.broadcast_to(base[:, None], (rows, LANES))

    mesh = plsc.VectorSubcoreMesh(core_axis_name="core", subcore_axis_name="subcore")
    cp = pltpu.CompilerParams()
    if "needs_layout_passes" in pltpu.CompilerParams.__dataclass_fields__:
        cp = dataclasses.replace(cp, needs_layout_passes=False)

    @pl.kernel(
        out_type=jax.ShapeDtypeStruct((n,), jnp.float32),
        mesh=mesh,
        scratch_types=[pltpu.VMEM((NUM_ACT * SIZE,), jnp.float32)],
        compiler_params=cp,
    )
    def spline_kernel(x_hbm, base_hbm, coeff_hbm, o_hbm, tab_v):
        pltpu.sync_copy(coeff_hbm, tab_v)

        def body(x_vmem, base_vmem, o_vmem):
            bvec = base_vmem[0, :]

            @pl.loop(0, CHUNK, step=LANES)
            def _(k):
                xv = x_vmem[pl.ds(k, LANES)]
                wv = xv * INV_GRID + jnp.float32(HALF)
                z = jnp.minimum(jnp.maximum(wv, 0.0), jnp.float32(2 * HALF - 1))
                i = z.astype(jnp.int32)
                frac = wv - i.astype(jnp.float32)
                idx = i + bvec
                c0 = plsc.load_gather(tab_v, [idx])
                c1 = plsc.load_gather(tab_v, [idx + 1])
                o_vmem[pl.ds(k, LANES)] = c0 + frac * (c1 - c0)

        pltpu.emit_pipeline(
            body,
            grid=(rows * CHUNKS_PER_ROW,),
            in_specs=[
                pl.BlockSpec((CHUNK,), lambda j: (j,)),
                pl.BlockSpec((1, LANES), lambda j: (j // CHUNKS_PER_ROW, 0)),
            ],
            out_specs=[pl.BlockSpec((CHUNK,), lambda j: (j,))],
            core_axis_name=("core", "subcore"),
            dimension_semantics=(pltpu.PARALLEL,),
        )(x_hbm, base_hbm, o_hbm)

    out = spline_kernel(x_flat, base_arr, coefficients_vect)
    return out.reshape(b, c, h, w)


# speedup vs baseline: 1095.6997x; 2.9898x over previous
import dataclasses

import jax
import jax.numpy as jnp
from jax.experimental import pallas as pl
from jax.experimental.pallas import tpu as pltpu
from jax.experimental.pallas import tpu_sc as plsc

NUM_ACT = 96
SIZE = 257
INV_GRID = 32.0
HALF = 128

ROW_LEN = 224 * 224
LANES = 16
CHUNK = 3584
CHUNKS_PER_ROW = ROW_LEN // CHUNK


def kernel(x, coefficients_vect):
    b, c, h, w = x.shape
    rows = b * c
    n = rows * ROW_LEN
    x_flat = x.reshape(n)
    base = (jnp.arange(rows, dtype=jnp.int32) % NUM_ACT) * SIZE
    base_arr = jnp.broadcast_to(base[:, None], (rows, LANES))

    mesh = plsc.VectorSubcoreMesh(core_axis_name="core", subcore_axis_name="subcore")
    cp = pltpu.CompilerParams()
    if "needs_layout_passes" in pltpu.CompilerParams.__dataclass_fields__:
        cp = dataclasses.replace(cp, needs_layout_passes=False)

    @pl.kernel(
        out_type=jax.ShapeDtypeStruct((n,), jnp.float32),
        mesh=mesh,
        scratch_types=[pltpu.VMEM((NUM_ACT * SIZE,), jnp.float32)],
        compiler_params=cp,
    )
    def spline_kernel(x_hbm, base_hbm, coeff_hbm, o_hbm, tab_v):
        pltpu.sync_copy(coeff_hbm, tab_v)

        def body(x_vmem, base_vmem, o_vmem):
            bvec = base_vmem[0, :]

            @plsc.parallel_loop(0, CHUNK, step=LANES, unroll=8)
            def _(k):
                xv = x_vmem[pl.ds(k, LANES)]
                wv = xv * INV_GRID + jnp.float32(HALF)
                z = jnp.minimum(jnp.maximum(wv, 0.0), jnp.float32(2 * HALF - 1))
                i = z.astype(jnp.int32)
                frac = wv - i.astype(jnp.float32)
                idx = i + bvec
                c0 = plsc.load_gather(tab_v, [idx])
                c1 = plsc.load_gather(tab_v, [idx + 1])
                o_vmem[pl.ds(k, LANES)] = c0 + frac * (c1 - c0)

        pltpu.emit_pipeline(
            body,
            grid=(rows * CHUNKS_PER_ROW,),
            in_specs=[
                pl.BlockSpec((CHUNK,), lambda j: (j,)),
                pl.BlockSpec((1, LANES), lambda j: (j // CHUNKS_PER_ROW, 0)),
            ],
            out_specs=[pl.BlockSpec((CHUNK,), lambda j: (j,))],
            core_axis_name=("core", "subcore"),
            dimension_semantics=(pltpu.PARALLEL,),
        )(x_hbm, base_hbm, o_hbm)

    out = spline_kernel(x_flat, base_arr, coefficients_vect)
    return out.reshape(b, c, h, w)


# speedup vs baseline: 2276.4413x; 2.0776x over previous
import dataclasses

import jax
import jax.numpy as jnp
from jax.experimental import pallas as pl
from jax.experimental.pallas import tpu as pltpu
from jax.experimental.pallas import tpu_sc as plsc

NUM_ACT = 96
SIZE = 257
INV_GRID = 32.0
HALF = 128

LANES = 16
BH = 56
BW = 128


def kernel(x, coefficients_vect):
    b, c, h, w = x.shape
    rows = b * c
    hb = h // BH
    wb = pl.cdiv(w, BW)
    base = (jnp.arange(rows, dtype=jnp.int32) % NUM_ACT) * SIZE
    base_arr = jnp.broadcast_to(base[:, None], (rows, LANES))

    mesh = plsc.VectorSubcoreMesh(core_axis_name="core", subcore_axis_name="subcore")
    cp = pltpu.CompilerParams(use_tc_tiling_on_sc=True)
    if "needs_layout_passes" in pltpu.CompilerParams.__dataclass_fields__:
        cp = dataclasses.replace(cp, needs_layout_passes=False)

    @pl.kernel(
        out_type=jax.ShapeDtypeStruct((b, c, h, w), jnp.float32),
        mesh=mesh,
        scratch_types=[pltpu.VMEM((NUM_ACT * SIZE,), jnp.float32)],
        compiler_params=cp,
    )
    def spline_kernel(x_hbm, base_hbm, coeff_hbm, o_hbm, tab_v):
        pltpu.sync_copy(coeff_hbm, tab_v)

        def body(x_vmem, base_vmem, o_vmem):
            bvec = base_vmem[0, :]

            @plsc.parallel_loop(0, BH * BW, step=LANES, unroll=8)
            def _(k):
                s = k // BW
                l = k % BW
                xv = x_vmem[0, 0, s, pl.ds(l, LANES)]
                wv = xv * INV_GRID + jnp.float32(HALF)
                z = jnp.minimum(jnp.maximum(wv, 0.0), jnp.float32(2 * HALF - 1))
                i = z.astype(jnp.int32)
                i = jnp.minimum(jnp.maximum(i, 0), 2 * HALF - 1)
                frac = wv - i.astype(jnp.float32)
                idx = i + bvec
                c0 = plsc.load_gather(tab_v, [idx])
                c1 = plsc.load_gather(tab_v, [idx + 1])
                o_vmem[0, 0, s, pl.ds(l, LANES)] = c0 + frac * (c1 - c0)

        pltpu.emit_pipeline(
            body,
            grid=(rows * hb * wb,),
            in_specs=[
                pl.BlockSpec(
                    (1, 1, BH, BW),
                    lambda j: (
                        j // (NUM_ACT * hb * wb),
                        (j // (hb * wb)) % NUM_ACT,
                        (j % (hb * wb)) // wb,
                        j % wb,
                    ),
                ),
                pl.BlockSpec((1, LANES), lambda j: (j // (hb * wb), 0)),
            ],
            out_specs=[
                pl.BlockSpec(
                    (1, 1, BH, BW),
                    lambda j: (
                        j // (NUM_ACT * hb * wb),
                        (j // (hb * wb)) % NUM_ACT,
                        (j % (hb * wb)) // wb,
                        j % wb,
                    ),
                )
            ],
            core_axis_name=("core", "subcore"),
            dimension_semantics=(pltpu.PARALLEL,),
        )(x_hbm, base_hbm, o_hbm)

    out = spline_kernel(x, base_arr, coefficients_vect)
    return out


# speedup vs baseline: 2794.1263x; 1.2274x over previous
import dataclasses

import jax
import jax.numpy as jnp
from jax.experimental import pallas as pl
from jax.experimental.pallas import tpu as pltpu
from jax.experimental.pallas import tpu_sc as plsc

NUM_ACT = 96
SIZE = 257
INV_GRID = 32.0
HALF = 128

LANES = 16
BH = 112
BW = 128


def kernel(x, coefficients_vect):
    b, c, h, w = x.shape
    rows = b * c
    hb = h // BH
    wb = pl.cdiv(w, BW)
    base = (jnp.arange(rows, dtype=jnp.int32) % NUM_ACT) * SIZE
    base_arr = jnp.broadcast_to(base[:, None], (rows, LANES))
    slope_vect = jnp.concatenate(
        [coefficients_vect[1:] - coefficients_vect[:-1],
         jnp.zeros((1,), jnp.float32)]
    )

    mesh = plsc.VectorSubcoreMesh(core_axis_name="core", subcore_axis_name="subcore")
    cp = pltpu.CompilerParams(use_tc_tiling_on_sc=True)
    if "needs_layout_passes" in pltpu.CompilerParams.__dataclass_fields__:
        cp = dataclasses.replace(cp, needs_layout_passes=False)

    @pl.kernel(
        out_type=jax.ShapeDtypeStruct((b, c, h, w), jnp.float32),
        mesh=mesh,
        scratch_types=[
            pltpu.VMEM((NUM_ACT * SIZE,), jnp.float32),
            pltpu.VMEM((NUM_ACT * SIZE,), jnp.float32),
        ],
        compiler_params=cp,
    )
    def spline_kernel(x_hbm, base_hbm, coeff_hbm, slope_hbm, o_hbm, tab_v, slp_v):
        pltpu.sync_copy(coeff_hbm, tab_v)
        pltpu.sync_copy(slope_hbm, slp_v)

        def body(x_vmem, base_vmem, o_vmem):
            bvec = base_vmem[0, :]

            @plsc.parallel_loop(0, BH * BW, step=LANES, unroll=8)
            def _(k):
                s = k // BW
                l = k % BW
                xv = x_vmem[0, 0, s, pl.ds(l, LANES)]
                wv = xv * INV_GRID + jnp.float32(HALF)
                i = wv.astype(jnp.int32)
                i = jnp.minimum(jnp.maximum(i, 0), 2 * HALF - 1)
                frac = wv - i.astype(jnp.float32)
                idx = i + bvec
                c0 = plsc.load_gather(tab_v, [idx])
                s0 = plsc.load_gather(slp_v, [idx])
                o_vmem[0, 0, s, pl.ds(l, LANES)] = c0 + frac * s0

        pltpu.emit_pipeline(
            body,
            grid=(rows * hb * wb,),
            in_specs=[
                pl.BlockSpec(
                    (1, 1, BH, BW),
                    lambda j: (
                        j // (NUM_ACT * hb * wb),
                        (j // (hb * wb)) % NUM_ACT,
                        (j % (hb * wb)) // wb,
                        j % wb,
                    ),
                ),
                pl.BlockSpec((1, LANES), lambda j: (j // (hb * wb), 0)),
            ],
            out_specs=[
                pl.BlockSpec(
                    (1, 1, BH, BW),
                    lambda j: (
                        j // (NUM_ACT * hb * wb),
                        (j // (hb * wb)) % NUM_ACT,
                        (j % (hb * wb)) // wb,
                        j % wb,
                    ),
                )
            ],
            core_axis_name=("core", "subcore"),
            dimension_semantics=(pltpu.PARALLEL,),
        )(x_hbm, base_hbm, o_hbm)

    out = spline_kernel(x, base_arr, coefficients_vect, slope_vect)
    return out
